# Initial kernel scaffold; baseline (speedup 1.0000x reference)
import jax
import jax.numpy as jnp
from jax.experimental import pallas as pl


def kernel(user_ids, movie_ids, title_idx, overview_idx, director_idx, cast_idx, genre_idx, prod_comp_idx, prod_count_idx, numeric_movie_data, user_W, title_W, overview_W, director_W, cast_W, genre_W, prod_comp_W, prod_count_W):
    raise NotImplementedError("write your pallas kernel here")


# speedup vs baseline: 1.5060x; 1.5060x over previous
import functools

import jax
import jax.numpy as jnp
from jax import lax
from jax.experimental import pallas as pl
from jax.experimental.pallas import tpu as pltpu
from jax.experimental.pallas import tpu_sc as plsc

_B = 16384
_NC, _NS = 2, 16
_NW = _NC * _NS
_CHUNK = _B // _NW
_S = 64
_NBLK = _CHUNK // _S

_L_OV, _L_CAST, _L_GENRE, _L_PC, _L_PCN = 20, 10, 5, 5, 3


def _pad_table(w, lpad, width, scale=None):
  v, c = w.shape
  if scale is not None:
    w = w * jnp.float32(scale)
  parts = []
  if lpad:
    parts.append(jnp.zeros((v, lpad), w.dtype))
  parts.append(w)
  rpad = width - lpad - c
  if rpad:
    parts.append(jnp.zeros((v, rpad), w.dtype))
  return jnp.concatenate(parts, axis=1) if len(parts) > 1 else w


def _expand_tokens(m_v, dst_v, L):
  lane = lax.iota(jnp.int32, 16)
  nv = -(-L // 16)

  def body(i, _):
    mL = m_v[pl.ds(i, 16)][0] * L
    for v in range(nv):
      dst_v[pl.ds(i * L + v * 16, 16)] = lane + (mL + v * 16)
    return 0

  lax.fori_loop(0, _S, body, 0)


def _sc_call(user_ids, movie_ids, title_idx, overview_idx_f, director_idx,
             cast_idx_f, genre_idx_f, prod_comp_idx_f, prod_count_idx_f,
             numeric_p, user_W, title_Wp, overview_Wp, director_Wp, cast_Wp,
             genre_Wp, prod_comp_Wp, prod_count_Wp):
  mesh = plsc.VectorSubcoreMesh(core_axis_name="c", subcore_axis_name="s",
                                num_cores=_NC, num_subcores=_NS)

  @functools.partial(
      pl.kernel,
      out_type=jax.ShapeDtypeStruct((_B,), jnp.float32),
      mesh=mesh,
      compiler_params=pltpu.CompilerParams(needs_layout_passes=False,
                                           use_tc_tiling_on_sc=False),
      scratch_types=[
          pltpu.VMEM((_S + 16,), jnp.int32),
          pltpu.VMEM((_S,), jnp.int32),
          pltpu.VMEM((_S,), jnp.int32),
          pltpu.VMEM((_S + 16,), jnp.int32),
          pltpu.VMEM((_S,), jnp.int32),
          pltpu.VMEM((_S,), jnp.int32),
          pltpu.VMEM((_S * _L_OV + 16,), jnp.int32),
          pltpu.VMEM((_S * _L_CAST + 16,), jnp.int32),
          pltpu.VMEM((_S * _L_GENRE + 16,), jnp.int32),
          pltpu.VMEM((_S * _L_PC + 16,), jnp.int32),
          pltpu.VMEM((_S * _L_PCN + 16,), jnp.int32),
          pltpu.VMEM((_S * _L_OV,), jnp.int32),
          pltpu.VMEM((_S * _L_CAST,), jnp.int32),
          pltpu.VMEM((_S * _L_GENRE,), jnp.int32),
          pltpu.VMEM((_S * _L_PC,), jnp.int32),
          pltpu.VMEM((_S * _L_PCN,), jnp.int32),
          pltpu.VMEM((_S, 400), jnp.float32),
          pltpu.VMEM((_S, 32), jnp.float32),
          pltpu.VMEM((_S * _L_OV, 32), jnp.float32),
          pltpu.VMEM((_S, 16), jnp.float32),
          pltpu.VMEM((_S * _L_CAST, 16), jnp.float32),
          pltpu.VMEM((_S * _L_GENRE, 32), jnp.float32),
          pltpu.VMEM((_S * _L_PC, 32), jnp.float32),
          pltpu.VMEM((_S * _L_PCN, 16), jnp.float32),
          pltpu.VMEM((_S, 16), jnp.float32),
          pltpu.VMEM((_S,), jnp.float32),
          pltpu.SemaphoreType.DMA,
      ],
  )
  def k(user_ids_h, movie_ids_h, title_idx_h, overview_idx_h, director_idx_h,
        cast_idx_h, genre_idx_h, prod_comp_idx_h, prod_count_idx_h,
        numeric_h, user_W_h, title_W_h, overview_W_h, director_W_h,
        cast_W_h, genre_W_h, prod_comp_W_h, prod_count_W_h, out_h,
        uids_v, mids_v, uq_v, m_v, tix_v, dix_v,
        ova_v, caa_v, gna_v, pca_v, pna_v,
        ovt_v, cat_v, gnt_v, pct_v, pnt_v,
        u_v, tit_v, ov_v, dir_v, ca_v, gn_v, pc_v, pn_v, nm_v,
        ob_v, sem):
    wid = lax.axis_index("c") * _NS + lax.axis_index("s")

    def block(blk, _):
      base = pl.multiple_of(wid * _CHUNK + blk * _S, _S)
      pltpu.sync_copy(user_ids_h.at[pl.ds(base, _S)], uids_v.at[pl.ds(0, _S)])
      pltpu.sync_copy(movie_ids_h.at[pl.ds(base, _S)], mids_v)
      for kk in range(_S // 16):
        m_v[pl.ds(kk * 16, 16)] = mids_v[pl.ds(kk * 16, 16)] - 1
        uq_v[pl.ds(kk * 16, 16)] = lax.shift_right_logical(
            uids_v[pl.ds(kk * 16, 16)], 2)

      m_sl = m_v.at[pl.ds(0, _S)]
      l1 = [
          pltpu.async_copy(user_W_h.at[uq_v], u_v, sem),
          pltpu.async_copy(title_idx_h.at[m_sl], tix_v, sem),
          pltpu.async_copy(director_idx_h.at[m_sl], dix_v, sem),
          pltpu.async_copy(numeric_h.at[m_sl], nm_v, sem),
      ]
      _expand_tokens(m_v, ova_v, _L_OV)
      _expand_tokens(m_v, caa_v, _L_CAST)
      _expand_tokens(m_v, gna_v, _L_GENRE)
      _expand_tokens(m_v, pca_v, _L_PC)
      _expand_tokens(m_v, pna_v, _L_PCN)
      l1 += [
          pltpu.async_copy(
              overview_idx_h.at[ova_v.at[pl.ds(0, _S * _L_OV)]], ovt_v, sem),
          pltpu.async_copy(
              cast_idx_h.at[caa_v.at[pl.ds(0, _S * _L_CAST)]], cat_v, sem),
          pltpu.async_copy(
              genre_idx_h.at[gna_v.at[pl.ds(0, _S * _L_GENRE)]], gnt_v, sem),
          pltpu.async_copy(
              prod_comp_idx_h.at[pca_v.at[pl.ds(0, _S * _L_PC)]], pct_v, sem),
          pltpu.async_copy(
              prod_count_idx_h.at[pna_v.at[pl.ds(0, _S * _L_PCN)]], pnt_v,
              sem),
      ]
      for d in l1:
        d.wait()

      l2 = [
          pltpu.async_copy(title_W_h.at[tix_v], tit_v, sem),
          pltpu.async_copy(director_W_h.at[dix_v], dir_v, sem),
          pltpu.async_copy(overview_W_h.at[ovt_v], ov_v, sem),
          pltpu.async_copy(cast_W_h.at[cat_v], ca_v, sem),
          pltpu.async_copy(genre_W_h.at[gnt_v], gn_v, sem),
          pltpu.async_copy(prod_comp_W_h.at[pct_v], pc_v, sem),
          pltpu.async_copy(prod_count_W_h.at[pnt_v], pn_v, sem),
      ]
      for d in l2:
        d.wait()

      lane = lax.iota(jnp.int32, 16)

      def sample(i, res):
        uoff = (uids_v[pl.ds(i, 16)][0] & 3) * 100
        u0 = u_v[i, pl.ds(uoff, 16)]
        u16 = u_v[i, pl.ds(uoff + 16, 16)]
        u32 = u_v[i, pl.ds(uoff + 32, 16)]
        u48 = u_v[i, pl.ds(uoff + 48, 16)]
        u64 = u_v[i, pl.ds(uoff + 64, 16)]
        u80 = u_v[i, pl.ds(uoff + 80, 16)]
        u84 = u_v[i, pl.ds(uoff + 84, 16)]

        acc0 = u0 * tit_v[i, pl.ds(0, 16)]
        acc1 = u16 * tit_v[i, pl.ds(16, 16)]

        ob = i * _L_OV
        s0 = ov_v[ob, pl.ds(0, 16)]
        s1 = ov_v[ob, pl.ds(16, 16)]
        for j in range(1, _L_OV):
          s0 = s0 + ov_v[ob + j, pl.ds(0, 16)]
          s1 = s1 + ov_v[ob + j, pl.ds(16, 16)]
        acc0 = acc0 + u16 * s0
        acc1 = acc1 + u32 * s1

        acc0 = acc0 + u32 * dir_v[i, pl.ds(0, 16)]

        cb = i * _L_CAST
        s0 = ca_v[cb, pl.ds(0, 16)]
        for j in range(1, _L_CAST):
          s0 = s0 + ca_v[cb + j, pl.ds(0, 16)]
        acc1 = acc1 + u48 * s0

        gb = i * _L_GENRE
        s0 = gn_v[gb, pl.ds(0, 16)]
        s1 = gn_v[gb, pl.ds(16, 16)]
        for j in range(1, _L_GENRE):
          s0 = s0 + gn_v[gb + j, pl.ds(0, 16)]
          s1 = s1 + gn_v[gb + j, pl.ds(16, 16)]
        acc0 = acc0 + u48 * s0
        acc1 = acc1 + u64 * s1

        pb = i * _L_PC
        s0 = pc_v[pb, pl.ds(0, 16)]
        s1 = pc_v[pb, pl.ds(16, 16)]
        for j in range(1, _L_PC):
          s0 = s0 + pc_v[pb + j, pl.ds(0, 16)]
          s1 = s1 + pc_v[pb + j, pl.ds(16, 16)]
        acc0 = acc0 + u64 * s0
        acc1 = acc1 + u80 * s1

        nb = i * _L_PCN
        s0 = pn_v[nb, pl.ds(0, 16)]
        for j in range(1, _L_PCN):
          s0 = s0 + pn_v[nb + j, pl.ds(0, 16)]
        acc0 = acc0 + u80 * s0

        acc1 = acc1 + u84 * nm_v[i, pl.ds(0, 16)]

        val = jnp.sum(acc0 + acc1)
        return jnp.where(lane == (i % 16), val, res)

      def group(g, _):
        res = lax.fori_loop(g * 16, (g + 1) * 16, sample,
                            lax.broadcast(jnp.float32(0.0), (16,)))
        ob_v[pl.ds(g * 16, 16)] = res
        return 0

      lax.fori_loop(0, _S // 16, group, 0)
      pltpu.sync_copy(ob_v, out_h.at[pl.ds(base, _S)])
      return 0

    lax.fori_loop(0, _NBLK, block, 0)

  return k(user_ids, movie_ids, title_idx, overview_idx_f, director_idx,
           cast_idx_f, genre_idx_f, prod_comp_idx_f, prod_count_idx_f,
           numeric_p, user_W, title_Wp, overview_Wp, director_Wp, cast_Wp,
           genre_Wp, prod_comp_Wp, prod_count_Wp)


def kernel(user_ids, movie_ids, title_idx, overview_idx, director_idx,
           cast_idx, genre_idx, prod_comp_idx, prod_count_idx,
           numeric_movie_data, user_W, title_W, overview_W, director_W,
           cast_W, genre_W, prod_comp_W, prod_count_W):
  title_Wp = _pad_table(title_W, 0, 32)
  overview_Wp = _pad_table(overview_W, 4, 32, scale=1.0 / _L_OV)
  director_Wp = _pad_table(director_W, 8, 16)
  cast_Wp = _pad_table(cast_W, 0, 16, scale=1.0 / _L_CAST)
  genre_Wp = _pad_table(genre_W, 10, 32, scale=1.0 / _L_GENRE)
  prod_comp_Wp = _pad_table(prod_comp_W, 9, 32, scale=1.0 / _L_PC)
  prod_count_Wp = _pad_table(prod_count_W, 3, 16, scale=1.0 / _L_PCN)
  numeric_p = _pad_table(numeric_movie_data, 9, 16)

  user_W4 = user_W.reshape(user_W.shape[0] // 4, 400)

  return _sc_call(user_ids, movie_ids, title_idx,
                  overview_idx.reshape(-1), director_idx,
                  cast_idx.reshape(-1), genre_idx.reshape(-1),
                  prod_comp_idx.reshape(-1), prod_count_idx.reshape(-1),
                  numeric_p, user_W4, title_Wp, overview_Wp, director_Wp,
                  cast_Wp, genre_Wp, prod_comp_Wp, prod_count_Wp)


# speedup vs baseline: 1.5067x; 1.0005x over previous
import functools

import jax
import jax.numpy as jnp
from jax import lax
from jax.experimental import pallas as pl
from jax.experimental.pallas import tpu as pltpu
from jax.experimental.pallas import tpu_sc as plsc

_B = 16384
_NC, _NS = 2, 16
_NW = _NC * _NS
_CHUNK = _B // _NW
_S = 64
_NBLK = _CHUNK // _S

_L_OV, _L_CAST, _L_GENRE, _L_PC, _L_PCN = 20, 10, 5, 5, 3


def _pad_table(w, lpad, width, scale=None):
  v, c = w.shape
  if scale is not None:
    w = w * jnp.float32(scale)
  parts = []
  if lpad:
    parts.append(jnp.zeros((v, lpad), w.dtype))
  parts.append(w)
  rpad = width - lpad - c
  if rpad:
    parts.append(jnp.zeros((v, rpad), w.dtype))
  return jnp.concatenate(parts, axis=1) if len(parts) > 1 else w


def _chunked_gather(src_h, idx_v, dst_v, n, sem, dmas):
  off = 0
  while off < n:
    c = min(128, n - off)
    dmas.append(
        pltpu.async_copy(src_h.at[idx_v.at[pl.ds(off, c)]],
                         dst_v.at[pl.ds(off, c)], sem))
    off += c


def _expand_tokens(m_v, dst_v, L):
  lane = lax.iota(jnp.int32, 16)
  nv = -(-L // 16)

  def body(i, _):
    mL = m_v[pl.ds(i, 16)][0] * L
    for v in range(nv):
      dst_v[pl.ds(i * L + v * 16, 16)] = lane + (mL + v * 16)
    return 0

  lax.fori_loop(0, _S, body, 0)


def _sc_call(user_ids, movie_ids, title_idx, overview_idx_f, director_idx,
             cast_idx_f, genre_idx_f, prod_comp_idx_f, prod_count_idx_f,
             numeric_p, user_W, title_Wp, overview_Wp, director_Wp, cast_Wp,
             genre_Wp, prod_comp_Wp, prod_count_Wp):
  mesh = plsc.VectorSubcoreMesh(core_axis_name="c", subcore_axis_name="s",
                                num_cores=_NC, num_subcores=_NS)

  @functools.partial(
      pl.kernel,
      out_type=jax.ShapeDtypeStruct((_B,), jnp.float32),
      mesh=mesh,
      compiler_params=pltpu.CompilerParams(needs_layout_passes=False,
                                           use_tc_tiling_on_sc=False),
      scratch_types=[
          pltpu.VMEM((_S + 16,), jnp.int32),
          pltpu.VMEM((_S,), jnp.int32),
          pltpu.VMEM((_S,), jnp.int32),
          pltpu.VMEM((_S + 16,), jnp.int32),
          pltpu.VMEM((_S,), jnp.int32),
          pltpu.VMEM((_S,), jnp.int32),
          pltpu.VMEM((_S * _L_OV + 16,), jnp.int32),
          pltpu.VMEM((_S * _L_CAST + 16,), jnp.int32),
          pltpu.VMEM((_S * _L_GENRE + 16,), jnp.int32),
          pltpu.VMEM((_S * _L_PC + 16,), jnp.int32),
          pltpu.VMEM((_S * _L_PCN + 16,), jnp.int32),
          pltpu.VMEM((_S * _L_OV,), jnp.int32),
          pltpu.VMEM((_S * _L_CAST,), jnp.int32),
          pltpu.VMEM((_S * _L_GENRE,), jnp.int32),
          pltpu.VMEM((_S * _L_PC,), jnp.int32),
          pltpu.VMEM((_S * _L_PCN,), jnp.int32),
          pltpu.VMEM((_S, 400), jnp.float32),
          pltpu.VMEM((_S, 32), jnp.float32),
          pltpu.VMEM((_S * _L_OV, 32), jnp.float32),
          pltpu.VMEM((_S, 16), jnp.float32),
          pltpu.VMEM((_S * _L_CAST, 16), jnp.float32),
          pltpu.VMEM((_S * _L_GENRE, 32), jnp.float32),
          pltpu.VMEM((_S * _L_PC, 32), jnp.float32),
          pltpu.VMEM((_S * _L_PCN, 16), jnp.float32),
          pltpu.VMEM((_S, 16), jnp.float32),
          pltpu.VMEM((_S,), jnp.float32),
          pltpu.SemaphoreType.DMA,
      ],
  )
  def k(user_ids_h, movie_ids_h, title_idx_h, overview_idx_h, director_idx_h,
        cast_idx_h, genre_idx_h, prod_comp_idx_h, prod_count_idx_h,
        numeric_h, user_W_h, title_W_h, overview_W_h, director_W_h,
        cast_W_h, genre_W_h, prod_comp_W_h, prod_count_W_h, out_h,
        uids_v, mids_v, uq_v, m_v, tix_v, dix_v,
        ova_v, caa_v, gna_v, pca_v, pna_v,
        ovt_v, cat_v, gnt_v, pct_v, pnt_v,
        u_v, tit_v, ov_v, dir_v, ca_v, gn_v, pc_v, pn_v, nm_v,
        ob_v, sem):
    wid = lax.axis_index("c") * _NS + lax.axis_index("s")

    def block(blk, _):
      base = pl.multiple_of(wid * _CHUNK + blk * _S, _S)
      pltpu.sync_copy(user_ids_h.at[pl.ds(base, _S)], uids_v.at[pl.ds(0, _S)])
      pltpu.sync_copy(movie_ids_h.at[pl.ds(base, _S)], mids_v)
      for kk in range(_S // 16):
        m_v[pl.ds(kk * 16, 16)] = mids_v[pl.ds(kk * 16, 16)] - 1
        uq_v[pl.ds(kk * 16, 16)] = lax.shift_right_logical(
            uids_v[pl.ds(kk * 16, 16)], 2)

      m_sl = m_v.at[pl.ds(0, _S)]
      l1 = [
          pltpu.async_copy(user_W_h.at[uq_v], u_v, sem),
          pltpu.async_copy(title_idx_h.at[m_sl], tix_v, sem),
          pltpu.async_copy(director_idx_h.at[m_sl], dix_v, sem),
          pltpu.async_copy(numeric_h.at[m_sl], nm_v, sem),
      ]
      _expand_tokens(m_v, ova_v, _L_OV)
      _expand_tokens(m_v, caa_v, _L_CAST)
      _expand_tokens(m_v, gna_v, _L_GENRE)
      _expand_tokens(m_v, pca_v, _L_PC)
      _expand_tokens(m_v, pna_v, _L_PCN)
      _chunked_gather(overview_idx_h, ova_v, ovt_v, _S * _L_OV, sem, l1)
      _chunked_gather(cast_idx_h, caa_v, cat_v, _S * _L_CAST, sem, l1)
      _chunked_gather(genre_idx_h, gna_v, gnt_v, _S * _L_GENRE, sem, l1)
      _chunked_gather(prod_comp_idx_h, pca_v, pct_v, _S * _L_PC, sem, l1)
      _chunked_gather(prod_count_idx_h, pna_v, pnt_v, _S * _L_PCN, sem, l1)
      for d in l1:
        d.wait()

      l2 = [
          pltpu.async_copy(title_W_h.at[tix_v], tit_v, sem),
          pltpu.async_copy(director_W_h.at[dix_v], dir_v, sem),
      ]
      _chunked_gather(overview_W_h, ovt_v, ov_v, _S * _L_OV, sem, l2)
      _chunked_gather(cast_W_h, cat_v, ca_v, _S * _L_CAST, sem, l2)
      _chunked_gather(genre_W_h, gnt_v, gn_v, _S * _L_GENRE, sem, l2)
      _chunked_gather(prod_comp_W_h, pct_v, pc_v, _S * _L_PC, sem, l2)
      _chunked_gather(prod_count_W_h, pnt_v, pn_v, _S * _L_PCN, sem, l2)
      for d in l2:
        d.wait()

      lane = lax.iota(jnp.int32, 16)

      def sample(i, res):
        uoff = (uids_v[pl.ds(i, 16)][0] & 3) * 100
        u0 = u_v[i, pl.ds(uoff, 16)]
        u16 = u_v[i, pl.ds(uoff + 16, 16)]
        u32 = u_v[i, pl.ds(uoff + 32, 16)]
        u48 = u_v[i, pl.ds(uoff + 48, 16)]
        u64 = u_v[i, pl.ds(uoff + 64, 16)]
        u80 = u_v[i, pl.ds(uoff + 80, 16)]
        u84 = u_v[i, pl.ds(uoff + 84, 16)]

        acc0 = u0 * tit_v[i, pl.ds(0, 16)]
        acc1 = u16 * tit_v[i, pl.ds(16, 16)]

        ob = i * _L_OV
        s0 = ov_v[ob, pl.ds(0, 16)]
        s1 = ov_v[ob, pl.ds(16, 16)]
        for j in range(1, _L_OV):
          s0 = s0 + ov_v[ob + j, pl.ds(0, 16)]
          s1 = s1 + ov_v[ob + j, pl.ds(16, 16)]
        acc0 = acc0 + u16 * s0
        acc1 = acc1 + u32 * s1

        acc0 = acc0 + u32 * dir_v[i, pl.ds(0, 16)]

        cb = i * _L_CAST
        s0 = ca_v[cb, pl.ds(0, 16)]
        for j in range(1, _L_CAST):
          s0 = s0 + ca_v[cb + j, pl.ds(0, 16)]
        acc1 = acc1 + u48 * s0

        gb = i * _L_GENRE
        s0 = gn_v[gb, pl.ds(0, 16)]
        s1 = gn_v[gb, pl.ds(16, 16)]
        for j in range(1, _L_GENRE):
          s0 = s0 + gn_v[gb + j, pl.ds(0, 16)]
          s1 = s1 + gn_v[gb + j, pl.ds(16, 16)]
        acc0 = acc0 + u48 * s0
        acc1 = acc1 + u64 * s1

        pb = i * _L_PC
        s0 = pc_v[pb, pl.ds(0, 16)]
        s1 = pc_v[pb, pl.ds(16, 16)]
        for j in range(1, _L_PC):
          s0 = s0 + pc_v[pb + j, pl.ds(0, 16)]
          s1 = s1 + pc_v[pb + j, pl.ds(16, 16)]
        acc0 = acc0 + u64 * s0
        acc1 = acc1 + u80 * s1

        nb = i * _L_PCN
        s0 = pn_v[nb, pl.ds(0, 16)]
        for j in range(1, _L_PCN):
          s0 = s0 + pn_v[nb + j, pl.ds(0, 16)]
        acc0 = acc0 + u80 * s0

        acc1 = acc1 + u84 * nm_v[i, pl.ds(0, 16)]

        val = jnp.sum(acc0 + acc1)
        return jnp.where(lane == (i % 16), val, res)

      def group(g, _):
        res = lax.fori_loop(g * 16, (g + 1) * 16, sample,
                            lax.broadcast(jnp.float32(0.0), (16,)))
        ob_v[pl.ds(g * 16, 16)] = res
        return 0

      lax.fori_loop(0, _S // 16, group, 0)
      pltpu.sync_copy(ob_v, out_h.at[pl.ds(base, _S)])
      return 0

    lax.fori_loop(0, _NBLK, block, 0)

  return k(user_ids, movie_ids, title_idx, overview_idx_f, director_idx,
           cast_idx_f, genre_idx_f, prod_comp_idx_f, prod_count_idx_f,
           numeric_p, user_W, title_Wp, overview_Wp, director_Wp, cast_Wp,
           genre_Wp, prod_comp_Wp, prod_count_Wp)


def kernel(user_ids, movie_ids, title_idx, overview_idx, director_idx,
           cast_idx, genre_idx, prod_comp_idx, prod_count_idx,
           numeric_movie_data, user_W, title_W, overview_W, director_W,
           cast_W, genre_W, prod_comp_W, prod_count_W):
  title_Wp = _pad_table(title_W, 0, 32)
  overview_Wp = _pad_table(overview_W, 4, 32, scale=1.0 / _L_OV)
  director_Wp = _pad_table(director_W, 8, 16)
  cast_Wp = _pad_table(cast_W, 0, 16, scale=1.0 / _L_CAST)
  genre_Wp = _pad_table(genre_W, 10, 32, scale=1.0 / _L_GENRE)
  prod_comp_Wp = _pad_table(prod_comp_W, 9, 32, scale=1.0 / _L_PC)
  prod_count_Wp = _pad_table(prod_count_W, 3, 16, scale=1.0 / _L_PCN)
  numeric_p = _pad_table(numeric_movie_data, 9, 16)

  user_W4 = user_W.reshape(user_W.shape[0] // 4, 400)

  return _sc_call(user_ids, movie_ids, title_idx,
                  overview_idx.reshape(-1), director_idx,
                  cast_idx.reshape(-1), genre_idx.reshape(-1),
                  prod_comp_idx.reshape(-1), prod_count_idx.reshape(-1),
                  numeric_p, user_W4, title_Wp, overview_Wp, director_Wp,
                  cast_Wp, genre_Wp, prod_comp_Wp, prod_count_Wp)


# speedup vs baseline: 1.5276x; 1.0138x over previous
import functools

import jax
import jax.numpy as jnp
from jax import lax
from jax.experimental import pallas as pl
from jax.experimental.pallas import tpu as pltpu
from jax.experimental.pallas import tpu_sc as plsc

_B = 16384
_NC, _NS = 2, 16
_NW = _NC * _NS
_CHUNK = _B // _NW
_S = 64
_NBLK = _CHUNK // _S

_L_OV, _L_CAST, _L_GENRE, _L_PC, _L_PCN = 20, 10, 5, 5, 3


_PAD_PLAN = (
    ("title", 0, 32, None, 5000),
    ("overview", 4, 32, 1.0 / _L_OV, 5000),
    ("director", 8, 16, None, 2000),
    ("cast", 0, 16, 1.0 / _L_CAST, 5000),
    ("genre", 10, 32, 1.0 / _L_GENRE, 20),
    ("prod_comp", 9, 32, 1.0 / _L_PC, 1000),
    ("prod_count", 3, 16, 1.0 / _L_PCN, 100),
    ("numeric", 9, 16, None, 10000),
)
_NSTEP = 10


def _pad_tables_tc(tables):
  in_specs, out_types, out_specs = [], [], []
  for (_, _, width, _, rows), t in zip(_PAD_PLAN, tables):
    v, c = t.shape
    if rows * _NSTEP == v:
      imap = lambda i: (i, 0)
    else:
      rows = v
      imap = lambda i: (0, 0)
    in_specs.append(pl.BlockSpec((rows, c), imap))
    out_types.append(jax.ShapeDtypeStruct((v, width), jnp.float32))
    out_specs.append(pl.BlockSpec((rows, width), imap))

  def body(*refs):
    ins, outs = refs[:len(tables)], refs[len(tables):]
    for (_, lpad, width, scale, _), x_ref, o_ref in zip(_PAD_PLAN, ins, outs):
      x = x_ref[...]
      if scale is not None:
        x = x * jnp.float32(scale)
      r, c = x.shape
      parts = []
      if lpad:
        parts.append(jnp.zeros((r, lpad), jnp.float32))
      parts.append(x)
      if width - lpad - c:
        parts.append(jnp.zeros((r, width - lpad - c), jnp.float32))
      o_ref[...] = jnp.concatenate(parts, axis=1)

  return pl.pallas_call(
      body,
      grid=(_NSTEP,),
      in_specs=in_specs,
      out_specs=out_specs,
      out_shape=out_types,
  )(*tables)


def _chunked_gather(src_h, idx_v, dst_v, n, sem, dmas):
  off = 0
  while off < n:
    c = min(128, n - off)
    dmas.append(
        pltpu.async_copy(src_h.at[idx_v.at[pl.ds(off, c)]],
                         dst_v.at[pl.ds(off, c)], sem))
    off += c


def _expand_tokens(m_v, dst_v, L):
  lane = lax.iota(jnp.int32, 16)
  nv = -(-L // 16)

  def body(i, _):
    mL = m_v[pl.ds(i, 16)][0] * L
    for v in range(nv):
      dst_v[pl.ds(i * L + v * 16, 16)] = lane + (mL + v * 16)
    return 0

  lax.fori_loop(0, _S, body, 0)


def _sc_call(user_ids, movie_ids, title_idx, overview_idx_f, director_idx,
             cast_idx_f, genre_idx_f, prod_comp_idx_f, prod_count_idx_f,
             numeric_p, user_W, title_Wp, overview_Wp, director_Wp, cast_Wp,
             genre_Wp, prod_comp_Wp, prod_count_Wp):
  mesh = plsc.VectorSubcoreMesh(core_axis_name="c", subcore_axis_name="s",
                                num_cores=_NC, num_subcores=_NS)

  @functools.partial(
      pl.kernel,
      out_type=jax.ShapeDtypeStruct((_B,), jnp.float32),
      mesh=mesh,
      compiler_params=pltpu.CompilerParams(needs_layout_passes=False,
                                           use_tc_tiling_on_sc=False),
      scratch_types=[
          pltpu.VMEM((_S + 16,), jnp.int32),
          pltpu.VMEM((_S,), jnp.int32),
          pltpu.VMEM((_S,), jnp.int32),
          pltpu.VMEM((_S + 16,), jnp.int32),
          pltpu.VMEM((_S,), jnp.int32),
          pltpu.VMEM((_S,), jnp.int32),
          pltpu.VMEM((_S * _L_OV + 16,), jnp.int32),
          pltpu.VMEM((_S * _L_CAST + 16,), jnp.int32),
          pltpu.VMEM((_S * _L_GENRE + 16,), jnp.int32),
          pltpu.VMEM((_S * _L_PC + 16,), jnp.int32),
          pltpu.VMEM((_S * _L_PCN + 16,), jnp.int32),
          pltpu.VMEM((_S * _L_OV,), jnp.int32),
          pltpu.VMEM((_S * _L_CAST,), jnp.int32),
          pltpu.VMEM((_S * _L_GENRE,), jnp.int32),
          pltpu.VMEM((_S * _L_PC,), jnp.int32),
          pltpu.VMEM((_S * _L_PCN,), jnp.int32),
          pltpu.VMEM((_S, 400), jnp.float32),
          pltpu.VMEM((_S, 32), jnp.float32),
          pltpu.VMEM((_S * _L_OV, 32), jnp.float32),
          pltpu.VMEM((_S, 16), jnp.float32),
          pltpu.VMEM((_S * _L_CAST, 16), jnp.float32),
          pltpu.VMEM((_S * _L_GENRE, 32), jnp.float32),
          pltpu.VMEM((_S * _L_PC, 32), jnp.float32),
          pltpu.VMEM((_S * _L_PCN, 16), jnp.float32),
          pltpu.VMEM((_S, 16), jnp.float32),
          pltpu.VMEM((_S,), jnp.float32),
          pltpu.SemaphoreType.DMA,
      ],
  )
  def k(user_ids_h, movie_ids_h, title_idx_h, overview_idx_h, director_idx_h,
        cast_idx_h, genre_idx_h, prod_comp_idx_h, prod_count_idx_h,
        numeric_h, user_W_h, title_W_h, overview_W_h, director_W_h,
        cast_W_h, genre_W_h, prod_comp_W_h, prod_count_W_h, out_h,
        uids_v, mids_v, uq_v, m_v, tix_v, dix_v,
        ova_v, caa_v, gna_v, pca_v, pna_v,
        ovt_v, cat_v, gnt_v, pct_v, pnt_v,
        u_v, tit_v, ov_v, dir_v, ca_v, gn_v, pc_v, pn_v, nm_v,
        ob_v, sem):
    wid = lax.axis_index("c") * _NS + lax.axis_index("s")

    def block(blk, _):
      base = pl.multiple_of(wid * _CHUNK + blk * _S, _S)
      pltpu.sync_copy(user_ids_h.at[pl.ds(base, _S)], uids_v.at[pl.ds(0, _S)])
      pltpu.sync_copy(movie_ids_h.at[pl.ds(base, _S)], mids_v)
      for kk in range(_S // 16):
        m_v[pl.ds(kk * 16, 16)] = mids_v[pl.ds(kk * 16, 16)] - 1
        uq_v[pl.ds(kk * 16, 16)] = lax.shift_right_logical(
            uids_v[pl.ds(kk * 16, 16)], 2)

      m_sl = m_v.at[pl.ds(0, _S)]
      l1 = [
          pltpu.async_copy(user_W_h.at[uq_v], u_v, sem),
          pltpu.async_copy(title_idx_h.at[m_sl], tix_v, sem),
          pltpu.async_copy(director_idx_h.at[m_sl], dix_v, sem),
          pltpu.async_copy(numeric_h.at[m_sl], nm_v, sem),
      ]
      _expand_tokens(m_v, ova_v, _L_OV)
      _expand_tokens(m_v, caa_v, _L_CAST)
      _expand_tokens(m_v, gna_v, _L_GENRE)
      _expand_tokens(m_v, pca_v, _L_PC)
      _expand_tokens(m_v, pna_v, _L_PCN)
      _chunked_gather(overview_idx_h, ova_v, ovt_v, _S * _L_OV, sem, l1)
      _chunked_gather(cast_idx_h, caa_v, cat_v, _S * _L_CAST, sem, l1)
      _chunked_gather(genre_idx_h, gna_v, gnt_v, _S * _L_GENRE, sem, l1)
      _chunked_gather(prod_comp_idx_h, pca_v, pct_v, _S * _L_PC, sem, l1)
      _chunked_gather(prod_count_idx_h, pna_v, pnt_v, _S * _L_PCN, sem, l1)
      for d in l1:
        d.wait()

      l2 = [
          pltpu.async_copy(title_W_h.at[tix_v], tit_v, sem),
          pltpu.async_copy(director_W_h.at[dix_v], dir_v, sem),
      ]
      _chunked_gather(overview_W_h, ovt_v, ov_v, _S * _L_OV, sem, l2)
      _chunked_gather(cast_W_h, cat_v, ca_v, _S * _L_CAST, sem, l2)
      _chunked_gather(genre_W_h, gnt_v, gn_v, _S * _L_GENRE, sem, l2)
      _chunked_gather(prod_comp_W_h, pct_v, pc_v, _S * _L_PC, sem, l2)
      _chunked_gather(prod_count_W_h, pnt_v, pn_v, _S * _L_PCN, sem, l2)
      for d in l2:
        d.wait()

      lane = lax.iota(jnp.int32, 16)

      def sample(i, res):
        uoff = (uids_v[pl.ds(i, 16)][0] & 3) * 100
        u0 = u_v[i, pl.ds(uoff, 16)]
        u16 = u_v[i, pl.ds(uoff + 16, 16)]
        u32 = u_v[i, pl.ds(uoff + 32, 16)]
        u48 = u_v[i, pl.ds(uoff + 48, 16)]
        u64 = u_v[i, pl.ds(uoff + 64, 16)]
        u80 = u_v[i, pl.ds(uoff + 80, 16)]
        u84 = u_v[i, pl.ds(uoff + 84, 16)]

        acc0 = u0 * tit_v[i, pl.ds(0, 16)]
        acc1 = u16 * tit_v[i, pl.ds(16, 16)]

        ob = i * _L_OV
        s0 = ov_v[ob, pl.ds(0, 16)]
        s1 = ov_v[ob, pl.ds(16, 16)]
        for j in range(1, _L_OV):
          s0 = s0 + ov_v[ob + j, pl.ds(0, 16)]
          s1 = s1 + ov_v[ob + j, pl.ds(16, 16)]
        acc0 = acc0 + u16 * s0
        acc1 = acc1 + u32 * s1

        acc0 = acc0 + u32 * dir_v[i, pl.ds(0, 16)]

        cb = i * _L_CAST
        s0 = ca_v[cb, pl.ds(0, 16)]
        for j in range(1, _L_CAST):
          s0 = s0 + ca_v[cb + j, pl.ds(0, 16)]
        acc1 = acc1 + u48 * s0

        gb = i * _L_GENRE
        s0 = gn_v[gb, pl.ds(0, 16)]
        s1 = gn_v[gb, pl.ds(16, 16)]
        for j in range(1, _L_GENRE):
          s0 = s0 + gn_v[gb + j, pl.ds(0, 16)]
          s1 = s1 + gn_v[gb + j, pl.ds(16, 16)]
        acc0 = acc0 + u48 * s0
        acc1 = acc1 + u64 * s1

        pb = i * _L_PC
        s0 = pc_v[pb, pl.ds(0, 16)]
        s1 = pc_v[pb, pl.ds(16, 16)]
        for j in range(1, _L_PC):
          s0 = s0 + pc_v[pb + j, pl.ds(0, 16)]
          s1 = s1 + pc_v[pb + j, pl.ds(16, 16)]
        acc0 = acc0 + u64 * s0
        acc1 = acc1 + u80 * s1

        nb = i * _L_PCN
        s0 = pn_v[nb, pl.ds(0, 16)]
        for j in range(1, _L_PCN):
          s0 = s0 + pn_v[nb + j, pl.ds(0, 16)]
        acc0 = acc0 + u80 * s0

        acc1 = acc1 + u84 * nm_v[i, pl.ds(0, 16)]

        val = jnp.sum(acc0 + acc1)
        return jnp.where(lane == (i % 16), val, res)

      def group(g, _):
        res = lax.fori_loop(g * 16, (g + 1) * 16, sample,
                            lax.broadcast(jnp.float32(0.0), (16,)))
        ob_v[pl.ds(g * 16, 16)] = res
        return 0

      lax.fori_loop(0, _S // 16, group, 0)
      pltpu.sync_copy(ob_v, out_h.at[pl.ds(base, _S)])
      return 0

    lax.fori_loop(0, _NBLK, block, 0)

  return k(user_ids, movie_ids, title_idx, overview_idx_f, director_idx,
           cast_idx_f, genre_idx_f, prod_comp_idx_f, prod_count_idx_f,
           numeric_p, user_W, title_Wp, overview_Wp, director_Wp, cast_Wp,
           genre_Wp, prod_comp_Wp, prod_count_Wp)


def kernel(user_ids, movie_ids, title_idx, overview_idx, director_idx,
           cast_idx, genre_idx, prod_comp_idx, prod_count_idx,
           numeric_movie_data, user_W, title_W, overview_W, director_W,
           cast_W, genre_W, prod_comp_W, prod_count_W):
  (title_Wp, overview_Wp, director_Wp, cast_Wp, genre_Wp, prod_comp_Wp,
   prod_count_Wp, numeric_p) = _pad_tables_tc(
       (title_W, overview_W, director_W, cast_W, genre_W, prod_comp_W,
        prod_count_W, numeric_movie_data))

  user_W4 = user_W.reshape(user_W.shape[0] // 4, 400)

  return _sc_call(user_ids, movie_ids, title_idx,
                  overview_idx.reshape(-1), director_idx,
                  cast_idx.reshape(-1), genre_idx.reshape(-1),
                  prod_comp_idx.reshape(-1), prod_count_idx.reshape(-1),
                  numeric_p, user_W4, title_Wp, overview_Wp, director_Wp,
                  cast_Wp, genre_Wp, prod_comp_Wp, prod_count_Wp)


# speedup vs baseline: 1.8217x; 1.1925x over previous
import functools

import jax
import jax.numpy as jnp
from jax import lax
from jax.experimental import pallas as pl
from jax.experimental.pallas import tpu as pltpu
from jax.experimental.pallas import tpu_sc as plsc

_B = 16384
_NC, _NS = 2, 16
_NW = _NC * _NS
_CHUNK = _B // _NW
_S = 64
_NBLK = _CHUNK // _S

_L_OV, _L_CAST, _L_GENRE, _L_PC, _L_PCN = 20, 10, 5, 5, 3


_PAD_PLAN = (
    ("title", 0, 32, None, 5000),
    ("overview", 4, 32, 1.0 / _L_OV, 5000),
    ("director", 8, 16, None, 2000),
    ("cast", 0, 16, 1.0 / _L_CAST, 5000),
    ("genre", 10, 32, 1.0 / _L_GENRE, 20),
    ("prod_comp", 9, 32, 1.0 / _L_PC, 1000),
    ("prod_count", 3, 16, 1.0 / _L_PCN, 100),
    ("numeric", 9, 16, None, 10000),
)
_NSTEP = 10


def _pad_tables_tc(tables):
  in_specs, out_types, out_specs = [], [], []
  for (_, _, width, _, rows), t in zip(_PAD_PLAN, tables):
    v, c = t.shape
    if rows * _NSTEP == v:
      imap = lambda i: (i, 0)
    else:
      rows = v
      imap = lambda i: (0, 0)
    in_specs.append(pl.BlockSpec((rows, c), imap))
    out_types.append(jax.ShapeDtypeStruct((v, width), jnp.float32))
    out_specs.append(pl.BlockSpec((rows, width), imap))

  def body(*refs):
    ins, outs = refs[:len(tables)], refs[len(tables):]
    for (_, lpad, width, scale, _), x_ref, o_ref in zip(_PAD_PLAN, ins, outs):
      x = x_ref[...]
      if scale is not None:
        x = x * jnp.float32(scale)
      r, c = x.shape
      parts = []
      if lpad:
        parts.append(jnp.zeros((r, lpad), jnp.float32))
      parts.append(x)
      if width - lpad - c:
        parts.append(jnp.zeros((r, width - lpad - c), jnp.float32))
      o_ref[...] = jnp.concatenate(parts, axis=1)

  return pl.pallas_call(
      body,
      grid=(_NSTEP,),
      in_specs=in_specs,
      out_specs=out_specs,
      out_shape=out_types,
  )(*tables)


def _chunked_gather(src_h, idx_v, dst_v, n, sem, dmas):
  off = 0
  while off < n:
    c = min(128, n - off)
    dmas.append(
        pltpu.async_copy(src_h.at[idx_v.at[pl.ds(off, c)]],
                         dst_v.at[pl.ds(off, c)], sem))
    off += c


def _expand_tokens(m_v, dst_v, L):
  lane = lax.iota(jnp.int32, 16)
  nv = -(-L // 16)

  def body(i, _):
    mL = m_v[pl.ds(i, 16)][0] * L
    for v in range(nv):
      dst_v[pl.ds(i * L + v * 16, 16)] = lane + (mL + v * 16)
    return 0

  lax.fori_loop(0, _S, body, 0)


def _sc_call(user_ids, movie_ids, title_idx, overview_idx_f, director_idx,
             cast_idx_f, genre_idx_f, prod_comp_idx_f, prod_count_idx_f,
             numeric_p, user_W, title_Wp, overview_Wp, director_Wp, cast_Wp,
             genre_Wp, prod_comp_Wp, prod_count_Wp):
  mesh = plsc.VectorSubcoreMesh(core_axis_name="c", subcore_axis_name="s",
                                num_cores=_NC, num_subcores=_NS)

  @functools.partial(
      pl.kernel,
      out_type=jax.ShapeDtypeStruct((_B,), jnp.float32),
      mesh=mesh,
      compiler_params=pltpu.CompilerParams(needs_layout_passes=False,
                                           use_tc_tiling_on_sc=False),
      scratch_types=[
          pltpu.VMEM((_S + 16,), jnp.int32),
          pltpu.VMEM((_S,), jnp.int32),
          pltpu.VMEM((_S + 16,), jnp.int32),
          pltpu.VMEM((_S,), jnp.int32),
          pltpu.VMEM((_S,), jnp.int32),
          pltpu.VMEM((_S * _L_OV + 16,), jnp.int32),
          pltpu.VMEM((_S * _L_CAST + 16,), jnp.int32),
          pltpu.VMEM((_S * _L_GENRE + 16,), jnp.int32),
          pltpu.VMEM((_S * _L_PC + 16,), jnp.int32),
          pltpu.VMEM((_S * _L_PCN + 16,), jnp.int32),
          pltpu.VMEM((_S * _L_OV,), jnp.int32),
          pltpu.VMEM((_S * _L_CAST,), jnp.int32),
          pltpu.VMEM((_S * _L_GENRE,), jnp.int32),
          pltpu.VMEM((_S * _L_PC,), jnp.int32),
          pltpu.VMEM((_S * _L_PCN,), jnp.int32),
          pltpu.VMEM((_S, 128), jnp.float32),
          pltpu.VMEM((_S, 32), jnp.float32),
          pltpu.VMEM((_S * _L_OV, 32), jnp.float32),
          pltpu.VMEM((_S, 16), jnp.float32),
          pltpu.VMEM((_S * _L_CAST, 16), jnp.float32),
          pltpu.VMEM((_S * _L_GENRE, 32), jnp.float32),
          pltpu.VMEM((_S * _L_PC, 32), jnp.float32),
          pltpu.VMEM((_S * _L_PCN, 16), jnp.float32),
          pltpu.VMEM((_S, 16), jnp.float32),
          pltpu.VMEM((_S,), jnp.float32),
          pltpu.SemaphoreType.DMA,
      ],
  )
  def k(user_ids_h, movie_ids_h, title_idx_h, overview_idx_h, director_idx_h,
        cast_idx_h, genre_idx_h, prod_comp_idx_h, prod_count_idx_h,
        numeric_h, user_W_h, title_W_h, overview_W_h, director_W_h,
        cast_W_h, genre_W_h, prod_comp_W_h, prod_count_W_h, out_h,
        uids_v, mids_v, m_v, tix_v, dix_v,
        ova_v, caa_v, gna_v, pca_v, pna_v,
        ovt_v, cat_v, gnt_v, pct_v, pnt_v,
        u_v, tit_v, ov_v, dir_v, ca_v, gn_v, pc_v, pn_v, nm_v,
        ob_v, sem):
    wid = lax.axis_index("c") * _NS + lax.axis_index("s")

    def block(blk, _):
      base = pl.multiple_of(wid * _CHUNK + blk * _S, _S)
      pltpu.sync_copy(user_ids_h.at[pl.ds(base, _S)], uids_v.at[pl.ds(0, _S)])
      pltpu.sync_copy(movie_ids_h.at[pl.ds(base, _S)], mids_v)
      for kk in range(_S // 16):
        m_v[pl.ds(kk * 16, 16)] = mids_v[pl.ds(kk * 16, 16)] - 1

      m_sl = m_v.at[pl.ds(0, _S)]
      l1 = [
          pltpu.async_copy(user_W_h.at[uids_v.at[pl.ds(0, _S)]], u_v, sem),
          pltpu.async_copy(title_idx_h.at[m_sl], tix_v, sem),
          pltpu.async_copy(director_idx_h.at[m_sl], dix_v, sem),
          pltpu.async_copy(numeric_h.at[m_sl], nm_v, sem),
      ]
      _expand_tokens(m_v, ova_v, _L_OV)
      _expand_tokens(m_v, caa_v, _L_CAST)
      _expand_tokens(m_v, gna_v, _L_GENRE)
      _expand_tokens(m_v, pca_v, _L_PC)
      _expand_tokens(m_v, pna_v, _L_PCN)
      _chunked_gather(overview_idx_h, ova_v, ovt_v, _S * _L_OV, sem, l1)
      _chunked_gather(cast_idx_h, caa_v, cat_v, _S * _L_CAST, sem, l1)
      _chunked_gather(genre_idx_h, gna_v, gnt_v, _S * _L_GENRE, sem, l1)
      _chunked_gather(prod_comp_idx_h, pca_v, pct_v, _S * _L_PC, sem, l1)
      _chunked_gather(prod_count_idx_h, pna_v, pnt_v, _S * _L_PCN, sem, l1)
      for d in l1:
        d.wait()

      l2 = [
          pltpu.async_copy(title_W_h.at[tix_v], tit_v, sem),
          pltpu.async_copy(director_W_h.at[dix_v], dir_v, sem),
      ]
      _chunked_gather(overview_W_h, ovt_v, ov_v, _S * _L_OV, sem, l2)
      _chunked_gather(cast_W_h, cat_v, ca_v, _S * _L_CAST, sem, l2)
      _chunked_gather(genre_W_h, gnt_v, gn_v, _S * _L_GENRE, sem, l2)
      _chunked_gather(prod_comp_W_h, pct_v, pc_v, _S * _L_PC, sem, l2)
      _chunked_gather(prod_count_W_h, pnt_v, pn_v, _S * _L_PCN, sem, l2)
      for d in l2:
        d.wait()

      lane = lax.iota(jnp.int32, 16)

      def sample(i, res):
        u0 = u_v[i, pl.ds(0, 16)]
        u16 = u_v[i, pl.ds(16, 16)]
        u32 = u_v[i, pl.ds(32, 16)]
        u48 = u_v[i, pl.ds(48, 16)]
        u64 = u_v[i, pl.ds(64, 16)]
        u80 = u_v[i, pl.ds(80, 16)]
        u84 = u_v[i, pl.ds(84, 16)]

        acc0 = u0 * tit_v[i, pl.ds(0, 16)]
        acc1 = u16 * tit_v[i, pl.ds(16, 16)]

        ob = i * _L_OV
        s0 = ov_v[ob, pl.ds(0, 16)]
        s1 = ov_v[ob, pl.ds(16, 16)]
        for j in range(1, _L_OV):
          s0 = s0 + ov_v[ob + j, pl.ds(0, 16)]
          s1 = s1 + ov_v[ob + j, pl.ds(16, 16)]
        acc0 = acc0 + u16 * s0
        acc1 = acc1 + u32 * s1

        acc0 = acc0 + u32 * dir_v[i, pl.ds(0, 16)]

        cb = i * _L_CAST
        s0 = ca_v[cb, pl.ds(0, 16)]
        for j in range(1, _L_CAST):
          s0 = s0 + ca_v[cb + j, pl.ds(0, 16)]
        acc1 = acc1 + u48 * s0

        gb = i * _L_GENRE
        s0 = gn_v[gb, pl.ds(0, 16)]
        s1 = gn_v[gb, pl.ds(16, 16)]
        for j in range(1, _L_GENRE):
          s0 = s0 + gn_v[gb + j, pl.ds(0, 16)]
          s1 = s1 + gn_v[gb + j, pl.ds(16, 16)]
        acc0 = acc0 + u48 * s0
        acc1 = acc1 + u64 * s1

        pb = i * _L_PC
        s0 = pc_v[pb, pl.ds(0, 16)]
        s1 = pc_v[pb, pl.ds(16, 16)]
        for j in range(1, _L_PC):
          s0 = s0 + pc_v[pb + j, pl.ds(0, 16)]
          s1 = s1 + pc_v[pb + j, pl.ds(16, 16)]
        acc0 = acc0 + u64 * s0
        acc1 = acc1 + u80 * s1

        nb = i * _L_PCN
        s0 = pn_v[nb, pl.ds(0, 16)]
        for j in range(1, _L_PCN):
          s0 = s0 + pn_v[nb + j, pl.ds(0, 16)]
        acc0 = acc0 + u80 * s0

        acc1 = acc1 + u84 * nm_v[i, pl.ds(0, 16)]

        val = jnp.sum(acc0 + acc1)
        return jnp.where(lane == (i % 16), val, res)

      def group(g, _):
        res = lax.fori_loop(g * 16, (g + 1) * 16, sample,
                            lax.broadcast(jnp.float32(0.0), (16,)))
        ob_v[pl.ds(g * 16, 16)] = res
        return 0

      lax.fori_loop(0, _S // 16, group, 0)
      pltpu.sync_copy(ob_v, out_h.at[pl.ds(base, _S)])
      return 0

    lax.fori_loop(0, _NBLK, block, 0)

  return k(user_ids, movie_ids, title_idx, overview_idx_f, director_idx,
           cast_idx_f, genre_idx_f, prod_comp_idx_f, prod_count_idx_f,
           numeric_p, user_W, title_Wp, overview_Wp, director_Wp, cast_Wp,
           genre_Wp, prod_comp_Wp, prod_count_Wp)


def kernel(user_ids, movie_ids, title_idx, overview_idx, director_idx,
           cast_idx, genre_idx, prod_comp_idx, prod_count_idx,
           numeric_movie_data, user_W, title_W, overview_W, director_W,
           cast_W, genre_W, prod_comp_W, prod_count_W):
  (title_Wp, overview_Wp, director_Wp, cast_Wp, genre_Wp, prod_comp_Wp,
   prod_count_Wp, numeric_p) = _pad_tables_tc(
       (title_W, overview_W, director_W, cast_W, genre_W, prod_comp_W,
        prod_count_W, numeric_movie_data))

  user_W128 = jnp.pad(user_W, ((0, 0), (0, 28)))

  return _sc_call(user_ids, movie_ids, title_idx,
                  overview_idx.reshape(-1), director_idx,
                  cast_idx.reshape(-1), genre_idx.reshape(-1),
                  prod_comp_idx.reshape(-1), prod_count_idx.reshape(-1),
                  numeric_p, user_W128, title_Wp, overview_Wp, director_Wp,
                  cast_Wp, genre_Wp, prod_comp_Wp, prod_count_Wp)


# speedup vs baseline: 2.5615x; 1.4061x over previous
import functools

import jax
import jax.numpy as jnp
from jax import lax
from jax.experimental import pallas as pl
from jax.experimental.pallas import tpu as pltpu
from jax.experimental.pallas import tpu_sc as plsc

_B = 16384
_NC, _NS = 2, 16
_NW = _NC * _NS
_CHUNK = _B // _NW
_S = 64
_NBLK = _CHUNK // _S

_L_OV, _L_CAST, _L_GENRE, _L_PC, _L_PCN = 20, 10, 5, 5, 3


_PAD_PLAN = (
    ("title", 0, 32, None, 5000),
    ("overview", 4, 32, 1.0 / _L_OV, 5000),
    ("director", 8, 16, None, 2000),
    ("cast", 0, 16, 1.0 / _L_CAST, 5000),
    ("genre", 10, 32, 1.0 / _L_GENRE, 20),
    ("prod_comp", 9, 32, 1.0 / _L_PC, 1000),
    ("prod_count", 3, 16, 1.0 / _L_PCN, 100),
    ("numeric", 9, 16, None, 10000),
)
_NSTEP = 10


def _pad_tables_tc(tables):
  in_specs, out_types, out_specs = [], [], []
  for (_, _, width, _, rows), t in zip(_PAD_PLAN, tables):
    v, c = t.shape
    if rows * _NSTEP == v:
      imap = lambda i: (i, 0)
    else:
      rows = v
      imap = lambda i: (0, 0)
    in_specs.append(pl.BlockSpec((rows, c), imap))
    out_types.append(jax.ShapeDtypeStruct((v, width), jnp.float32))
    out_specs.append(pl.BlockSpec((rows, width), imap))

  def body(*refs):
    ins, outs = refs[:len(tables)], refs[len(tables):]
    for (_, lpad, width, scale, _), x_ref, o_ref in zip(_PAD_PLAN, ins, outs):
      x = x_ref[...]
      if scale is not None:
        x = x * jnp.float32(scale)
      r, c = x.shape
      parts = []
      if lpad:
        parts.append(jnp.zeros((r, lpad), jnp.float32))
      parts.append(x)
      if width - lpad - c:
        parts.append(jnp.zeros((r, width - lpad - c), jnp.float32))
      o_ref[...] = jnp.concatenate(parts, axis=1)

  return pl.pallas_call(
      body,
      grid=(_NSTEP,),
      in_specs=in_specs,
      out_specs=out_specs,
      out_shape=out_types,
  )(*tables)


def _pad_user_tc(user_W):
  n = user_W.shape[0]
  rows = 8000

  def body(x_ref, o_ref):
    o_ref[...] = jnp.concatenate(
        [x_ref[...], jnp.zeros((rows, 28), jnp.float32)], axis=1)

  return pl.pallas_call(
      body,
      grid=(n // rows,),
      in_specs=[pl.BlockSpec((rows, 100), lambda i: (i, 0))],
      out_specs=pl.BlockSpec((rows, 128), lambda i: (i, 0)),
      out_shape=jax.ShapeDtypeStruct((n, 128), jnp.float32),
  )(user_W)


def _chunked_gather(src_h, idx_v, dst_v, n, sem, dmas):
  off = 0
  while off < n:
    c = min(128, n - off)
    dmas.append(
        pltpu.async_copy(src_h.at[idx_v.at[pl.ds(off, c)]],
                         dst_v.at[pl.ds(off, c)], sem))
    off += c


def _expand_tokens(m_v, dst_v, L):
  lane = lax.iota(jnp.int32, 16)
  nv = -(-L // 16)

  def body(i, _):
    mL = m_v[pl.ds(i, 16)][0] * L
    for v in range(nv):
      dst_v[pl.ds(i * L + v * 16, 16)] = lane + (mL + v * 16)
    return 0

  lax.fori_loop(0, _S, body, 0)


def _sc_call(user_ids, movie_ids, title_idx, overview_idx_f, director_idx,
             cast_idx_f, genre_idx_f, prod_comp_idx_f, prod_count_idx_f,
             numeric_p, user_W, title_Wp, overview_Wp, director_Wp, cast_Wp,
             genre_Wp, prod_comp_Wp, prod_count_Wp):
  mesh = plsc.VectorSubcoreMesh(core_axis_name="c", subcore_axis_name="s",
                                num_cores=_NC, num_subcores=_NS)

  @functools.partial(
      pl.kernel,
      out_type=jax.ShapeDtypeStruct((_B,), jnp.float32),
      mesh=mesh,
      compiler_params=pltpu.CompilerParams(needs_layout_passes=False,
                                           use_tc_tiling_on_sc=False),
      scratch_types=[
          pltpu.VMEM((_S + 16,), jnp.int32),
          pltpu.VMEM((_S,), jnp.int32),
          pltpu.VMEM((_S + 16,), jnp.int32),
          pltpu.VMEM((_S,), jnp.int32),
          pltpu.VMEM((_S,), jnp.int32),
          pltpu.VMEM((_S * _L_OV + 16,), jnp.int32),
          pltpu.VMEM((_S * _L_CAST + 16,), jnp.int32),
          pltpu.VMEM((_S * _L_GENRE + 16,), jnp.int32),
          pltpu.VMEM((_S * _L_PC + 16,), jnp.int32),
          pltpu.VMEM((_S * _L_PCN + 16,), jnp.int32),
          pltpu.VMEM((_S * _L_OV,), jnp.int32),
          pltpu.VMEM((_S * _L_CAST,), jnp.int32),
          pltpu.VMEM((_S * _L_GENRE,), jnp.int32),
          pltpu.VMEM((_S * _L_PC,), jnp.int32),
          pltpu.VMEM((_S * _L_PCN,), jnp.int32),
          pltpu.VMEM((_S, 128), jnp.float32),
          pltpu.VMEM((_S, 32), jnp.float32),
          pltpu.VMEM((_S * _L_OV, 32), jnp.float32),
          pltpu.VMEM((_S, 16), jnp.float32),
          pltpu.VMEM((_S * _L_CAST, 16), jnp.float32),
          pltpu.VMEM((_S * _L_GENRE, 32), jnp.float32),
          pltpu.VMEM((_S * _L_PC, 32), jnp.float32),
          pltpu.VMEM((_S * _L_PCN, 16), jnp.float32),
          pltpu.VMEM((_S, 16), jnp.float32),
          pltpu.VMEM((_S,), jnp.float32),
          pltpu.SemaphoreType.DMA,
      ],
  )
  def k(user_ids_h, movie_ids_h, title_idx_h, overview_idx_h, director_idx_h,
        cast_idx_h, genre_idx_h, prod_comp_idx_h, prod_count_idx_h,
        numeric_h, user_W_h, title_W_h, overview_W_h, director_W_h,
        cast_W_h, genre_W_h, prod_comp_W_h, prod_count_W_h, out_h,
        uids_v, mids_v, m_v, tix_v, dix_v,
        ova_v, caa_v, gna_v, pca_v, pna_v,
        ovt_v, cat_v, gnt_v, pct_v, pnt_v,
        u_v, tit_v, ov_v, dir_v, ca_v, gn_v, pc_v, pn_v, nm_v,
        ob_v, sem):
    wid = lax.axis_index("c") * _NS + lax.axis_index("s")

    def block(blk, _):
      base = pl.multiple_of(wid * _CHUNK + blk * _S, _S)
      pltpu.sync_copy(user_ids_h.at[pl.ds(base, _S)], uids_v.at[pl.ds(0, _S)])
      pltpu.sync_copy(movie_ids_h.at[pl.ds(base, _S)], mids_v)
      for kk in range(_S // 16):
        m_v[pl.ds(kk * 16, 16)] = mids_v[pl.ds(kk * 16, 16)] - 1

      m_sl = m_v.at[pl.ds(0, _S)]
      l1 = [
          pltpu.async_copy(user_W_h.at[uids_v.at[pl.ds(0, _S)]], u_v, sem),
          pltpu.async_copy(title_idx_h.at[m_sl], tix_v, sem),
          pltpu.async_copy(director_idx_h.at[m_sl], dix_v, sem),
          pltpu.async_copy(numeric_h.at[m_sl], nm_v, sem),
      ]
      _expand_tokens(m_v, ova_v, _L_OV)
      _expand_tokens(m_v, caa_v, _L_CAST)
      _expand_tokens(m_v, gna_v, _L_GENRE)
      _expand_tokens(m_v, pca_v, _L_PC)
      _expand_tokens(m_v, pna_v, _L_PCN)
      _chunked_gather(overview_idx_h, ova_v, ovt_v, _S * _L_OV, sem, l1)
      _chunked_gather(cast_idx_h, caa_v, cat_v, _S * _L_CAST, sem, l1)
      _chunked_gather(genre_idx_h, gna_v, gnt_v, _S * _L_GENRE, sem, l1)
      _chunked_gather(prod_comp_idx_h, pca_v, pct_v, _S * _L_PC, sem, l1)
      _chunked_gather(prod_count_idx_h, pna_v, pnt_v, _S * _L_PCN, sem, l1)
      for d in l1:
        d.wait()

      l2 = [
          pltpu.async_copy(title_W_h.at[tix_v], tit_v, sem),
          pltpu.async_copy(director_W_h.at[dix_v], dir_v, sem),
      ]
      _chunked_gather(overview_W_h, ovt_v, ov_v, _S * _L_OV, sem, l2)
      _chunked_gather(cast_W_h, cat_v, ca_v, _S * _L_CAST, sem, l2)
      _chunked_gather(genre_W_h, gnt_v, gn_v, _S * _L_GENRE, sem, l2)
      _chunked_gather(prod_comp_W_h, pct_v, pc_v, _S * _L_PC, sem, l2)
      _chunked_gather(prod_count_W_h, pnt_v, pn_v, _S * _L_PCN, sem, l2)
      for d in l2:
        d.wait()

      lane = lax.iota(jnp.int32, 16)

      def sample(i, res):
        u0 = u_v[i, pl.ds(0, 16)]
        u16 = u_v[i, pl.ds(16, 16)]
        u32 = u_v[i, pl.ds(32, 16)]
        u48 = u_v[i, pl.ds(48, 16)]
        u64 = u_v[i, pl.ds(64, 16)]
        u80 = u_v[i, pl.ds(80, 16)]
        u84 = u_v[i, pl.ds(84, 16)]

        acc0 = u0 * tit_v[i, pl.ds(0, 16)]
        acc1 = u16 * tit_v[i, pl.ds(16, 16)]

        ob = i * _L_OV
        s0 = ov_v[ob, pl.ds(0, 16)]
        s1 = ov_v[ob, pl.ds(16, 16)]
        for j in range(1, _L_OV):
          s0 = s0 + ov_v[ob + j, pl.ds(0, 16)]
          s1 = s1 + ov_v[ob + j, pl.ds(16, 16)]
        acc0 = acc0 + u16 * s0
        acc1 = acc1 + u32 * s1

        acc0 = acc0 + u32 * dir_v[i, pl.ds(0, 16)]

        cb = i * _L_CAST
        s0 = ca_v[cb, pl.ds(0, 16)]
        for j in range(1, _L_CAST):
          s0 = s0 + ca_v[cb + j, pl.ds(0, 16)]
        acc1 = acc1 + u48 * s0

        gb = i * _L_GENRE
        s0 = gn_v[gb, pl.ds(0, 16)]
        s1 = gn_v[gb, pl.ds(16, 16)]
        for j in range(1, _L_GENRE):
          s0 = s0 + gn_v[gb + j, pl.ds(0, 16)]
          s1 = s1 + gn_v[gb + j, pl.ds(16, 16)]
        acc0 = acc0 + u48 * s0
        acc1 = acc1 + u64 * s1

        pb = i * _L_PC
        s0 = pc_v[pb, pl.ds(0, 16)]
        s1 = pc_v[pb, pl.ds(16, 16)]
        for j in range(1, _L_PC):
          s0 = s0 + pc_v[pb + j, pl.ds(0, 16)]
          s1 = s1 + pc_v[pb + j, pl.ds(16, 16)]
        acc0 = acc0 + u64 * s0
        acc1 = acc1 + u80 * s1

        nb = i * _L_PCN
        s0 = pn_v[nb, pl.ds(0, 16)]
        for j in range(1, _L_PCN):
          s0 = s0 + pn_v[nb + j, pl.ds(0, 16)]
        acc0 = acc0 + u80 * s0

        acc1 = acc1 + u84 * nm_v[i, pl.ds(0, 16)]

        val = jnp.sum(acc0 + acc1)
        return jnp.where(lane == (i % 16), val, res)

      def group(g, _):
        res = lax.fori_loop(g * 16, (g + 1) * 16, sample,
                            lax.broadcast(jnp.float32(0.0), (16,)))
        ob_v[pl.ds(g * 16, 16)] = res
        return 0

      lax.fori_loop(0, _S // 16, group, 0)
      pltpu.sync_copy(ob_v, out_h.at[pl.ds(base, _S)])
      return 0

    lax.fori_loop(0, _NBLK, block, 0)

  return k(user_ids, movie_ids, title_idx, overview_idx_f, director_idx,
           cast_idx_f, genre_idx_f, prod_comp_idx_f, prod_count_idx_f,
           numeric_p, user_W, title_Wp, overview_Wp, director_Wp, cast_Wp,
           genre_Wp, prod_comp_Wp, prod_count_Wp)


def kernel(user_ids, movie_ids, title_idx, overview_idx, director_idx,
           cast_idx, genre_idx, prod_comp_idx, prod_count_idx,
           numeric_movie_data, user_W, title_W, overview_W, director_W,
           cast_W, genre_W, prod_comp_W, prod_count_W):
  (title_Wp, overview_Wp, director_Wp, cast_Wp, genre_Wp, prod_comp_Wp,
   prod_count_Wp, numeric_p) = _pad_tables_tc(
       (title_W, overview_W, director_W, cast_W, genre_W, prod_comp_W,
        prod_count_W, numeric_movie_data))

  user_W128 = _pad_user_tc(user_W)

  return _sc_call(user_ids, movie_ids, title_idx,
                  overview_idx.reshape(-1), director_idx,
                  cast_idx.reshape(-1), genre_idx.reshape(-1),
                  prod_comp_idx.reshape(-1), prod_count_idx.reshape(-1),
                  numeric_p, user_W128, title_Wp, overview_Wp, director_Wp,
                  cast_Wp, genre_Wp, prod_comp_Wp, prod_count_Wp)
